# baseline (speedup 1.0000x reference)
import functools

import jax
import jax.numpy as jnp
from jax import lax
from jax.experimental import pallas as pl
from jax.experimental.pallas import tpu as pltpu
from jax.experimental.pallas import tpu_sc as plsc

N = 10000
E = 320000
D_IN = 128
D_HID = 128
D_OUT = 64

NC, NS = 2, 16
NW = NC * NS
CHUNK = 64
NBUF = 5
CPT = 160
NCHUNK = CPT * NW
E_PAD = NCHUNK * CHUNK
R_TOT = E_PAD // 128
RPS = R_TOT // NS

N_PAD16 = 10240
ZROWS = N_PAD16 // NS
DEG_PAD = 10240
JUNK = N

_MESH = dict(core_axis_name="c", subcore_axis_name="s", num_cores=NC,
             num_subcores=NS)


@functools.partial(
    pl.kernel,
    out_type=jax.ShapeDtypeStruct((DEG_PAD,), jnp.float32),
    mesh=plsc.VectorSubcoreMesh(**_MESH),
    scratch_types=[
        pltpu.VMEM_SHARED((DEG_PAD,), jnp.float32),
        pltpu.VMEM((RPS, 128), jnp.int32),
        pltpu.VMEM((128,), jnp.float32),
        pltpu.SemaphoreType.DMA,
    ],
)
def _deg_kernel(dst2d, zeros640, ones128, deg_out, deg_sh, dst_all, onesv,
                sem):
    cid = lax.axis_index("c")
    sid = lax.axis_index("s")

    @pl.when(cid == 0)
    def _():
        sl = pl.ds(sid * 640, 640)
        pltpu.sync_copy(zeros640, deg_sh.at[sl])
        pltpu.sync_copy(ones128, onesv)
        pltpu.sync_copy(dst2d.at[pl.ds(sid * RPS, RPS)], dst_all)
        plsc.subcore_barrier()

        def fire(r, carry):
            pltpu.async_copy(onesv, deg_sh.at[dst_all.at[r]], sem, add=True)
            return carry

        lax.fori_loop(0, RPS, fire, 0)

        def drain(r, carry):
            pltpu.make_async_copy(ones128, onesv, sem).wait()
            return carry

        lax.fori_loop(0, RPS, drain, 0)
        plsc.subcore_barrier()
        pltpu.sync_copy(deg_sh.at[sl], deg_out.at[sl])


def _make_agg_split(DS):
    CPT2 = NCHUNK // NS

    @functools.partial(
        pl.kernel,
        out_type=(jax.ShapeDtypeStruct((N_PAD16, DS), jnp.float32),
                  jax.ShapeDtypeStruct((N_PAD16, DS), jnp.float32)),
        mesh=plsc.VectorSubcoreMesh(**_MESH),
        compiler_params=pltpu.CompilerParams(use_tc_tiling_on_sc=False),
        scratch_types=[
            pltpu.VMEM_SHARED((N_PAD16, DS), jnp.float32),
            pltpu.VMEM_SHARED((N_PAD16, DS), jnp.float32),
            [pltpu.VMEM((2, CHUNK), jnp.int32)] * NBUF,
            [pltpu.VMEM((CHUNK, DS), jnp.float32)] * NBUF,
            [pltpu.SemaphoreType.DMA] * NBUF,
            [pltpu.SemaphoreType.DMA] * NBUF,
            [pltpu.SemaphoreType.DMA] * NBUF,
        ],
    )
    def agg(idx3d, scl, scr, zrows, out_a, out_b, stage_sh, agg_sh, idxv,
            rows, isem, gsem, ssem):
        cid = lax.axis_index("c")
        sid = lax.axis_index("s")
        g_base = sid * CPT2
        sl = pl.ds(sid * ZROWS, ZROWS)
        pltpu.sync_copy(zrows, agg_sh.at[sl])

        @pl.when(cid == 0)
        def _():
            pltpu.sync_copy(scl.at[sl], stage_sh.at[sl])

        @pl.when(cid == 1)
        def _():
            pltpu.sync_copy(scr.at[sl], stage_sh.at[sl])

        plsc.subcore_barrier()

        def idx_load(g, b):
            pltpu.async_copy(idx3d.at[g_base + g], idxv[b], isem[b])

        def idx_wait(b):
            pltpu.make_async_copy(idx3d.at[0], idxv[b], isem[b]).wait()

        def gather_start(b):
            pltpu.async_copy(stage_sh.at[idxv[b].at[0]], rows[b], gsem[b])

        def gather_wait(b):
            pltpu.make_async_copy(scl.at[pl.ds(0, CHUNK)], rows[b],
                                  gsem[b]).wait()

        def scatter_start(b):
            pltpu.async_copy(rows[b], agg_sh.at[idxv[b].at[1]], ssem[b],
                             add=True)

        def scatter_wait(b):
            pltpu.make_async_copy(scl.at[pl.ds(0, CHUNK)], rows[b],
                                  ssem[b]).wait()

        for b in range(NBUF):
            idx_load(b, b)
        for b in range(NBUF):
            idx_wait(b)
            gather_start(b)

        def body(i, carry):
            g0 = i * NBUF
            for b in range(NBUF):
                gather_wait(b)
                scatter_start(b)

            @pl.when(i < CPT2 // NBUF - 1)
            def _():
                for b in range(NBUF):
                    scatter_wait(b)
                    idx_load(g0 + NBUF + b, b)
                for b in range(NBUF):
                    idx_wait(b)
                    gather_start(b)

            return carry

        lax.fori_loop(0, CPT2 // NBUF, body, 0)
        for b in range(NBUF):
            scatter_wait(b)
        plsc.subcore_barrier()

        @pl.when(cid == 0)
        def _():
            pltpu.sync_copy(agg_sh.at[sl], out_a.at[sl])

        @pl.when(cid == 1)
        def _():
            pltpu.sync_copy(agg_sh.at[sl], out_b.at[sl])

    return agg


_agg_l1 = _make_agg_split(D_HID // 2)
_agg_l2 = _make_agg_split(D_OUT // 2)

_RB = 1000


def _lin1_body(deg_ref, x_ref, w_ref, ol_ref, or_ref):
    dis = lax.rsqrt(deg_ref[...] + 1.0)
    xw = dis * jnp.dot(x_ref[...], w_ref[...],
                       preferred_element_type=jnp.float32)
    ol_ref[...] = xw[:, :D_HID // 2]
    or_ref[...] = xw[:, D_HID // 2:]


def _lin1(deg_col, x, W1):
    half = pl.BlockSpec((_RB, D_HID // 2), lambda i: (i, 0))
    return pl.pallas_call(
        _lin1_body,
        grid=(N // _RB,),
        in_specs=[
            pl.BlockSpec((_RB, 1), lambda i: (i, 0)),
            pl.BlockSpec((_RB, D_IN), lambda i: (i, 0)),
            pl.BlockSpec((D_IN, D_HID), lambda i: (0, 0)),
        ],
        out_specs=(half, half),
        out_shape=(jax.ShapeDtypeStruct((N_PAD16, D_HID // 2), jnp.float32),
                   jax.ShapeDtypeStruct((N_PAD16, D_HID // 2), jnp.float32)),
    )(deg_col, x, W1)


def _lin2_body(deg_ref, al_ref, ar_ref, sl_ref, sr_ref, b1_ref, w2_ref,
               ol_ref, or_ref):
    dis = lax.rsqrt(deg_ref[...] + 1.0)
    tot = jnp.concatenate([al_ref[...] + sl_ref[...],
                           ar_ref[...] + sr_ref[...]], axis=1)
    h1 = jnp.maximum(dis * tot + b1_ref[...], 0.0)
    s2 = dis * jnp.dot(h1, w2_ref[...], preferred_element_type=jnp.float32)
    ol_ref[...] = s2[:, :D_OUT // 2]
    or_ref[...] = s2[:, D_OUT // 2:]


def _lin2(deg_col, a1l, a1r, s1l, s1r, b1_row, W2):
    halfin = pl.BlockSpec((_RB, D_HID // 2), lambda i: (i, 0))
    halfout = pl.BlockSpec((_RB, D_OUT // 2), lambda i: (i, 0))
    return pl.pallas_call(
        _lin2_body,
        grid=(N // _RB,),
        in_specs=[
            pl.BlockSpec((_RB, 1), lambda i: (i, 0)),
            halfin, halfin, halfin, halfin,
            pl.BlockSpec((1, D_HID), lambda i: (0, 0)),
            pl.BlockSpec((D_HID, D_OUT), lambda i: (0, 0)),
        ],
        out_specs=(halfout, halfout),
        out_shape=(jax.ShapeDtypeStruct((N_PAD16, D_OUT // 2), jnp.float32),
                   jax.ShapeDtypeStruct((N_PAD16, D_OUT // 2), jnp.float32)),
    )(deg_col, a1l, a1r, s1l, s1r, b1_row, W2)


def _h2_body(deg_ref, al_ref, ar_ref, sl_ref, sr_ref, b2_ref, o_ref):
    dis = lax.rsqrt(deg_ref[...] + 1.0)
    tot = jnp.concatenate([al_ref[...] + sl_ref[...],
                           ar_ref[...] + sr_ref[...]], axis=1)
    o_ref[...] = dis * tot + b2_ref[...]


def _h2(deg_col, a2l, a2r, s2l, s2r, b2_row):
    halfin = pl.BlockSpec((_RB, D_OUT // 2), lambda i: (i, 0))
    return pl.pallas_call(
        _h2_body,
        grid=(N // _RB,),
        in_specs=[
            pl.BlockSpec((_RB, 1), lambda i: (i, 0)),
            halfin, halfin, halfin, halfin,
            pl.BlockSpec((1, D_OUT), lambda i: (0, 0)),
        ],
        out_specs=pl.BlockSpec((_RB, D_OUT), lambda i: (i, 0)),
        out_shape=jax.ShapeDtypeStruct((N, D_OUT), jnp.float32),
    )(deg_col, a2l, a2r, s2l, s2r, b2_row)


_DB = 400


def _dec_body(hi_ref, hf_ref, o_ref):
    p = lax.dot_general(hi_ref[...], hf_ref[...], (((1,), (1,)), ((), ())),
                        preferred_element_type=jnp.float32)
    o_ref[...] = jax.nn.sigmoid(p)


def _decode(h2):
    return pl.pallas_call(
        _dec_body,
        grid=(N // _DB,),
        in_specs=[
            pl.BlockSpec((_DB, D_OUT), lambda i: (i, 0)),
            pl.BlockSpec((N, D_OUT), lambda i: (0, 0)),
        ],
        out_specs=pl.BlockSpec((_DB, N), lambda i: (i, 0)),
        out_shape=jax.ShapeDtypeStruct((N, N), jnp.float32),
    )(h2, h2)


def kernel(x, edge_index, W1, b1, W2, b2):
    src = edge_index[0].astype(jnp.int32)
    dst = edge_index[1].astype(jnp.int32)
    pad = E_PAD - E
    pad_src = (jnp.arange(pad, dtype=jnp.int32) * 2003) % N
    pad_dst = jnp.full((pad,), JUNK, jnp.int32)
    src_p = jnp.concatenate([src, pad_src])
    dst_p = jnp.concatenate([dst, pad_dst])
    idx3d = jnp.stack([src_p.reshape(NCHUNK, CHUNK),
                       dst_p.reshape(NCHUNK, CHUNK)], axis=1)
    dst2d = dst_p.reshape(R_TOT, 128)

    zeros640 = jnp.zeros((640,), jnp.float32)
    ones128 = jnp.ones((128,), jnp.float32)
    zrows_h = jnp.zeros((ZROWS, D_HID // 2), jnp.float32)
    zrows_o = jnp.zeros((ZROWS, D_OUT // 2), jnp.float32)

    deg_pad = _deg_kernel(dst2d, zeros640, ones128)
    deg_col = deg_pad[:N].reshape(N, 1)

    s1l, s1r = _lin1(deg_col, x, W1)
    a1l, a1r = _agg_l1(idx3d, s1l, s1r, zrows_h)
    s2l, s2r = _lin2(deg_col, a1l, a1r, s1l, s1r, b1.reshape(1, D_HID), W2)
    a2l, a2r = _agg_l2(idx3d, s2l, s2r, zrows_o)
    h2 = _h2(deg_col, a2l, a2r, s2l, s2r, b2.reshape(1, D_OUT))
    return _decode(h2)


# speedup vs baseline: 1.1000x; 1.1000x over previous
import functools

import jax
import jax.numpy as jnp
from jax import lax
from jax.experimental import pallas as pl
from jax.experimental.pallas import tpu as pltpu
from jax.experimental.pallas import tpu_sc as plsc

N = 10000
E = 320000
D_IN = 128
D_HID = 128
D_OUT = 64

NC, NS = 2, 16
NW = NC * NS
CHUNK = 64
NBUF = 5
CPT = 160
NCHUNK = CPT * NW
E_PAD = NCHUNK * CHUNK
R_TOT = E_PAD // 128
RPS = R_TOT // NS

N_PAD16 = 10240
ZROWS = N_PAD16 // NS
DEG_PAD = 10240
JUNK = N

_MESH = dict(core_axis_name="c", subcore_axis_name="s", num_cores=NC,
             num_subcores=NS)


_DEG_ROWS = E // 128


@functools.partial(
    pl.kernel,
    out_type=jax.ShapeDtypeStruct((DEG_PAD,), jnp.float32),
    mesh=plsc.VectorSubcoreMesh(**_MESH),
    scratch_types=[
        pltpu.VMEM_SHARED((DEG_PAD,), jnp.float32),
        pltpu.VMEM((160, 128), jnp.int32),
        pltpu.VMEM((128,), jnp.float32),
        pltpu.SemaphoreType.DMA,
    ],
)
def _deg_kernel(dst2d, zeros640, ones128, deg_out, deg_sh, dst_all, onesv,
                sem):
    cid = lax.axis_index("c")
    sid = lax.axis_index("s")

    @pl.when(cid == 0)
    def _():
        sl = pl.ds(sid * 640, 640)
        pltpu.sync_copy(zeros640, deg_sh.at[sl])
        pltpu.sync_copy(ones128, onesv)

        @pl.when(sid < 15)
        def _():
            pltpu.sync_copy(dst2d.at[pl.ds(sid * 160, 160)], dst_all)

        @pl.when(sid == 15)
        def _():
            pltpu.sync_copy(dst2d.at[pl.ds(2400, _DEG_ROWS - 2400)],
                            dst_all.at[pl.ds(0, _DEG_ROWS - 2400)])

        plsc.subcore_barrier()

        def fire(r, carry):
            pltpu.async_copy(onesv, deg_sh.at[dst_all.at[r]], sem, add=True)
            return carry

        def drain(r, carry):
            pltpu.make_async_copy(ones128, onesv, sem).wait()
            return carry

        @pl.when(sid < 15)
        def _():
            lax.fori_loop(0, 160, fire, 0)
            lax.fori_loop(0, 160, drain, 0)

        @pl.when(sid == 15)
        def _():
            lax.fori_loop(0, _DEG_ROWS - 2400, fire, 0)
            lax.fori_loop(0, _DEG_ROWS - 2400, drain, 0)

        plsc.subcore_barrier()
        pltpu.sync_copy(deg_sh.at[sl], deg_out.at[sl])


def _make_agg(D, tc_tiling=True):
    @functools.partial(
        pl.kernel,
        out_type=(jax.ShapeDtypeStruct((N_PAD16, D), jnp.float32),
                  jax.ShapeDtypeStruct((N_PAD16, D), jnp.float32)),
        mesh=plsc.VectorSubcoreMesh(**_MESH),
        compiler_params=pltpu.CompilerParams(use_tc_tiling_on_sc=tc_tiling),
        scratch_types=[
            pltpu.VMEM_SHARED((N_PAD16, D), jnp.float32),
            [pltpu.VMEM((2, CHUNK), jnp.int32)] * NBUF,
            [pltpu.VMEM((CHUNK, D), jnp.float32)] * NBUF,
            [pltpu.SemaphoreType.DMA] * NBUF,
            [pltpu.SemaphoreType.DMA] * NBUF,
            [pltpu.SemaphoreType.DMA] * NBUF,
        ],
    )
    def agg(idx3d, scaled, zrows, out_a, out_b, agg_sh, idxv, rows, isem,
            gsem, ssem):
        cid = lax.axis_index("c")
        sid = lax.axis_index("s")
        wid = sid * NC + cid
        g_base = wid * CPT
        sl = pl.ds(sid * ZROWS, ZROWS)
        pltpu.sync_copy(zrows, agg_sh.at[sl])
        plsc.subcore_barrier()

        def idx_load(g, b):
            pltpu.async_copy(idx3d.at[g_base + g], idxv[b], isem[b])

        def idx_wait(b):
            pltpu.make_async_copy(idx3d.at[0], idxv[b], isem[b]).wait()

        def gather_start(b):
            pltpu.async_copy(scaled.at[idxv[b].at[0]], rows[b], gsem[b])

        def gather_wait(b):
            pltpu.make_async_copy(scaled.at[pl.ds(0, CHUNK)], rows[b],
                                  gsem[b]).wait()

        def scatter_start(b):
            pltpu.async_copy(rows[b], agg_sh.at[idxv[b].at[1]], ssem[b],
                             add=True)

        def scatter_wait(b):
            pltpu.make_async_copy(scaled.at[pl.ds(0, CHUNK)], rows[b],
                                  ssem[b]).wait()

        for b in range(NBUF):
            idx_load(b, b)
        for b in range(NBUF):
            idx_wait(b)
            gather_start(b)

        def body(i, carry):
            g0 = i * NBUF
            for b in range(NBUF):
                gather_wait(b)
                scatter_start(b)

            @pl.when(i < CPT // NBUF - 1)
            def _():
                for b in range(NBUF):
                    scatter_wait(b)
                    idx_load(g0 + NBUF + b, b)
                for b in range(NBUF):
                    idx_wait(b)
                    gather_start(b)

            return carry

        lax.fori_loop(0, CPT // NBUF, body, 0)
        for b in range(NBUF):
            scatter_wait(b)
        plsc.subcore_barrier()

        @pl.when(cid == 0)
        def _():
            pltpu.sync_copy(agg_sh.at[sl], out_a.at[sl])

        @pl.when(cid == 1)
        def _():
            pltpu.sync_copy(agg_sh.at[sl], out_b.at[sl])

    return agg


_agg128 = _make_agg(D_HID)
_agg64 = _make_agg(D_OUT, tc_tiling=False)

_RB = 1000


def _lin1_body(deg_ref, x_ref, w_ref, o_ref):
    dis = lax.rsqrt(deg_ref[...] + 1.0)
    xw = jnp.dot(x_ref[...], w_ref[...], preferred_element_type=jnp.float32)
    o_ref[...] = dis * xw


def _lin1(deg_col, x, W1):
    return pl.pallas_call(
        _lin1_body,
        grid=(N // _RB,),
        in_specs=[
            pl.BlockSpec((_RB, 1), lambda i: (i, 0)),
            pl.BlockSpec((_RB, D_IN), lambda i: (i, 0)),
            pl.BlockSpec((D_IN, D_HID), lambda i: (0, 0)),
        ],
        out_specs=pl.BlockSpec((_RB, D_HID), lambda i: (i, 0)),
        out_shape=jax.ShapeDtypeStruct((N, D_HID), jnp.float32),
    )(deg_col, x, W1)


def _lin2_body(deg_ref, aa_ref, ab_ref, s1_ref, b1_ref, w2_ref, o_ref):
    dis = lax.rsqrt(deg_ref[...] + 1.0)
    h1 = dis * (aa_ref[...] + ab_ref[...] + s1_ref[...]) + b1_ref[...]
    h1 = jnp.maximum(h1, 0.0)
    o_ref[...] = dis * jnp.dot(h1, w2_ref[...],
                               preferred_element_type=jnp.float32)


def _lin2(deg_col, agg_a, agg_b, scaled1, b1_row, W2):
    return pl.pallas_call(
        _lin2_body,
        grid=(N // _RB,),
        in_specs=[
            pl.BlockSpec((_RB, 1), lambda i: (i, 0)),
            pl.BlockSpec((_RB, D_HID), lambda i: (i, 0)),
            pl.BlockSpec((_RB, D_HID), lambda i: (i, 0)),
            pl.BlockSpec((_RB, D_HID), lambda i: (i, 0)),
            pl.BlockSpec((1, D_HID), lambda i: (0, 0)),
            pl.BlockSpec((D_HID, D_OUT), lambda i: (0, 0)),
        ],
        out_specs=pl.BlockSpec((_RB, D_OUT), lambda i: (i, 0)),
        out_shape=jax.ShapeDtypeStruct((N, D_OUT), jnp.float32),
    )(deg_col, agg_a, agg_b, scaled1, b1_row, W2)


def _h2_body(deg_ref, aa_ref, ab_ref, s2_ref, b2_ref, o_ref):
    dis = lax.rsqrt(deg_ref[...] + 1.0)
    tot = (aa_ref[...] + ab_ref[...] + s2_ref[...])
    o_ref[...] = dis * tot + b2_ref[...]


def _h2(deg_col, agg_a, agg_b, scaled2, b2_row):
    return pl.pallas_call(
        _h2_body,
        grid=(N // _RB,),
        in_specs=[
            pl.BlockSpec((_RB, 1), lambda i: (i, 0)),
            pl.BlockSpec((_RB, D_OUT), lambda i: (i, 0)),
            pl.BlockSpec((_RB, D_OUT), lambda i: (i, 0)),
            pl.BlockSpec((_RB, D_OUT), lambda i: (i, 0)),
            pl.BlockSpec((1, D_OUT), lambda i: (0, 0)),
        ],
        out_specs=pl.BlockSpec((_RB, D_OUT), lambda i: (i, 0)),
        out_shape=jax.ShapeDtypeStruct((N, D_OUT), jnp.float32),
    )(deg_col, agg_a, agg_b, scaled2, b2_row)


_DB = 400


def _dec_body(hi_ref, hf_ref, o_ref):
    p = lax.dot_general(hi_ref[...], hf_ref[...], (((1,), (1,)), ((), ())),
                        preferred_element_type=jnp.float32)
    o_ref[...] = jax.nn.sigmoid(p)


def _decode(h2):
    return pl.pallas_call(
        _dec_body,
        grid=(N // _DB,),
        in_specs=[
            pl.BlockSpec((_DB, D_OUT), lambda i: (i, 0)),
            pl.BlockSpec((N, D_OUT), lambda i: (0, 0)),
        ],
        out_specs=pl.BlockSpec((_DB, N), lambda i: (i, 0)),
        out_shape=jax.ShapeDtypeStruct((N, N), jnp.float32),
    )(h2, h2)


def kernel(x, edge_index, W1, b1, W2, b2):
    src = edge_index[0].astype(jnp.int32)
    dst = edge_index[1].astype(jnp.int32)
    pad = E_PAD - E
    pad_src = (jnp.arange(pad, dtype=jnp.int32) * 2003) % N
    pad_dst = jnp.full((pad,), JUNK, jnp.int32)
    src_p = jnp.concatenate([src, pad_src])
    dst_p = jnp.concatenate([dst, pad_dst])
    idx3d = jnp.stack([src_p.reshape(NCHUNK, CHUNK),
                       dst_p.reshape(NCHUNK, CHUNK)], axis=1)
    dst2d = dst.reshape(_DEG_ROWS, 128)

    zeros640 = jnp.zeros((640,), jnp.float32)
    ones128 = jnp.ones((128,), jnp.float32)
    zrows_h = jnp.zeros((ZROWS, D_HID), jnp.float32)
    zrows_o = jnp.zeros((ZROWS, D_OUT), jnp.float32)

    deg_pad = _deg_kernel(dst2d, zeros640, ones128)
    deg_col = deg_pad[:N].reshape(N, 1)

    scaled1 = _lin1(deg_col, x, W1)
    a1a, a1b = _agg128(idx3d, scaled1, zrows_h)
    scaled2 = _lin2(deg_col, a1a, a1b, scaled1, b1.reshape(1, D_HID), W2)
    a2a, a2b = _agg64(idx3d, scaled2, zrows_o)
    h2 = _h2(deg_col, a2a, a2b, scaled2, b2.reshape(1, D_OUT))
    return _decode(h2)
